# baseline (speedup 1.0000x reference)
import functools

import jax
import jax.numpy as jnp
from jax import lax
from jax.experimental import pallas as pl
from jax.experimental.pallas import tpu as pltpu
from jax.experimental.pallas import tpu_sc as plsc

N_FIELDS = 100
BATCH = 16384

_NC = 2
_NS = 16
_NW = _NC * _NS
_B_PER_W = BATCH // _NW


def _make_lookup():
    mesh = plsc.VectorSubcoreMesh(core_axis_name="c", subcore_axis_name="s")

    @functools.partial(
        pl.kernel,
        mesh=mesh,
        out_type=jax.ShapeDtypeStruct((BATCH,), jnp.float32),
        scratch_types=[
            pltpu.VMEM((_B_PER_W,), jnp.int32),
            pltpu.VMEM((_B_PER_W,), jnp.float32),
            pltpu.SemaphoreType.DMA,
        ],
        compiler_params=pltpu.CompilerParams(
            needs_layout_passes=False, skip_device_barrier=True
        ),
    )
    def lookup(idx_hbm, tab_hbm, out_hbm, idx_v, out_v, sem):
        wid = lax.axis_index("s") * _NC + lax.axis_index("c")
        base = wid * _B_PER_W
        pltpu.sync_copy(idx_hbm.at[pl.ds(base, _B_PER_W)], idx_v)
        pltpu.async_copy(tab_hbm.at[idx_v], out_v, sem).wait()
        pltpu.sync_copy(out_v, out_hbm.at[pl.ds(base, _B_PER_W)])

    return lookup


_lookup = _make_lookup()


def kernel(id_f, zpt):
    out = _lookup(id_f.astype(jnp.int32), zpt)
    return out[:, None]


# speedup vs baseline: 4.9458x; 4.9458x over previous
import functools

import jax
import jax.numpy as jnp
from jax import lax
from jax.experimental import pallas as pl
from jax.experimental.pallas import tpu as pltpu
from jax.experimental.pallas import tpu_sc as plsc

N_FIELDS = 100
BATCH = 16384

_NC = 1
_NS = 16
_L = 16
_NW = _NC * _NS
_B_PER_W = BATCH // _NW


def _make_lookup():
    mesh = plsc.VectorSubcoreMesh(
        core_axis_name="c", subcore_axis_name="s", num_cores=_NC
    )

    @functools.partial(
        pl.kernel,
        mesh=mesh,
        out_type=jax.ShapeDtypeStruct((BATCH,), jnp.float32),
        scratch_types=[
            pltpu.VMEM((_B_PER_W,), jnp.int32),
            pltpu.VMEM((128,), jnp.float32),
            pltpu.VMEM((_B_PER_W,), jnp.float32),
        ],
        compiler_params=pltpu.CompilerParams(
            needs_layout_passes=False, skip_device_barrier=True
        ),
    )
    def lookup(idx_hbm, tab_hbm, out_hbm, idx_v, tab_v, out_v):
        wid = lax.axis_index("s") * _NC + lax.axis_index("c")
        base = wid * _B_PER_W
        pltpu.sync_copy(idx_hbm.at[pl.ds(base, _B_PER_W)], idx_v)
        pltpu.sync_copy(tab_hbm, tab_v.at[pl.ds(0, N_FIELDS)])
        for i in range(_B_PER_W // _L):
            idx = idx_v[pl.ds(i * _L, _L)]
            out_v[pl.ds(i * _L, _L)] = plsc.load_gather(tab_v, [idx])
        pltpu.sync_copy(out_v, out_hbm.at[pl.ds(base, _B_PER_W)])

    return lookup


_lookup = _make_lookup()


def kernel(id_f, zpt):
    out = _lookup(id_f.astype(jnp.int32), zpt)
    return out[:, None]


# speedup vs baseline: 5.2495x; 1.0614x over previous
import functools

import jax
import jax.numpy as jnp
from jax import lax
from jax.experimental import pallas as pl
from jax.experimental.pallas import tpu as pltpu
from jax.experimental.pallas import tpu_sc as plsc

N_FIELDS = 100
BATCH = 16384

_NC = 1
_NS = 16
_L = 16
_NW = _NC * _NS
_B_PER_W = BATCH // _NW


def _make_lookup():
    mesh = plsc.VectorSubcoreMesh(
        core_axis_name="c", subcore_axis_name="s", num_cores=_NC
    )

    @functools.partial(
        pl.kernel,
        mesh=mesh,
        out_type=jax.ShapeDtypeStruct((BATCH,), jnp.float32),
        scratch_types=[
            pltpu.VMEM((_B_PER_W,), jnp.int32),
            pltpu.VMEM((128,), jnp.float32),
            pltpu.VMEM((_B_PER_W,), jnp.float32),
            pltpu.SemaphoreType.DMA,
            pltpu.SemaphoreType.DMA,
        ],
        compiler_params=pltpu.CompilerParams(
            needs_layout_passes=False, skip_device_barrier=True
        ),
    )
    def lookup(idx_hbm, tab_hbm, out_hbm, idx_v, tab_v, out_v, sem_i, sem_t):
        wid = lax.axis_index("s") * _NC + lax.axis_index("c")
        base = wid * _B_PER_W
        ci = pltpu.async_copy(idx_hbm.at[pl.ds(base, _B_PER_W)], idx_v, sem_i)
        ct = pltpu.async_copy(tab_hbm, tab_v.at[pl.ds(0, N_FIELDS)], sem_t)
        ct.wait()
        ci.wait()

        @plsc.parallel_loop(0, _B_PER_W, _L, unroll=4)
        def _gather(off):
            s = pl.ds(off, _L)
            out_v[s] = plsc.load_gather(tab_v, [idx_v[s]])

        pltpu.sync_copy(out_v, out_hbm.at[pl.ds(base, _B_PER_W)])

    return lookup


_lookup = _make_lookup()


def kernel(id_f, zpt):
    out = _lookup(id_f.astype(jnp.int32), zpt)
    return out[:, None]
